# baseline (speedup 1.0000x reference)
import functools

import jax
import jax.numpy as jnp
from jax import lax
from jax.experimental import pallas as pl
from jax.experimental.pallas import tpu as pltpu
from jax.experimental.pallas import tpu_sc as plsc

N = 50000
N_PAD = 50048
B = 32
E = 800000
K = 2
NC = 2
NS = 16
NW = NC * NS
CH = 128
E_PAD = 819200
EW = E_PAD // NW
NCH = EW // CH
RPT = N_PAD // NS
DUMP = N
DW = 16
ZR = 782
IB = 20
NB = NCH // IB
NBUF = 4
ZRS = 184
RB = 3128

_MESH = plsc.VectorSubcoreMesh(core_axis_name="c", subcore_axis_name="s")
_SC_PARAMS = pltpu.CompilerParams(use_tc_tiling_on_sc=False)


def _sc_agg_body(x_hbm, src_hbm, dst_hbm, out_hbm, sidx, didx, rows0, rows1,
                 rows2, rows3, zrow, acc, gsem0, gsem1, gsem2, gsem3,
                 ssem0, ssem1, ssem2, ssem3):
    rows = (rows0, rows1, rows2, rows3)
    gsem = (gsem0, gsem1, gsem2, gsem3)
    ssem = (ssem0, ssem1, ssem2, ssem3)
    c = lax.axis_index("c")
    s = lax.axis_index("s")
    w = c * NS + s

    z16 = jnp.zeros((16,), jnp.float32)

    def zfill(i, carry):
        zrow[i, 0:16] = z16
        zrow[i, 16:32] = z16
        return carry

    lax.fori_loop(0, ZRS, zfill, 0)

    base = s * RPT
    for j in range(RPT // ZRS):
        pltpu.sync_copy(zrow, acc.at[pl.ds(base + j * ZRS, ZRS)])

    plsc.subcore_barrier()

    def batch(bi, carry):
        pltpu.sync_copy(src_hbm.at[pl.ds(w * NCH + bi * IB, IB)], sidx)
        pltpu.sync_copy(dst_hbm.at[pl.ds(w * NCH + bi * IB, IB)], didx)

        for b in range(NBUF):
            pltpu.async_copy(x_hbm.at[sidx.at[b]], rows[b], gsem[b])

        def group(g, carry2):
            c = g * NBUF
            for b in range(NBUF):
                pltpu.make_async_copy(
                    x_hbm.at[sidx.at[c + b]], rows[b], gsem[b]).wait()
                pltpu.async_copy(
                    rows[b], acc.at[didx.at[c + b]], ssem[b], add=True)
            for b in range(NBUF):
                pltpu.make_async_copy(
                    rows[b], acc.at[didx.at[c + b]], ssem[b]).wait()
                pltpu.async_copy(
                    x_hbm.at[sidx.at[c + NBUF + b]], rows[b], gsem[b])
            return carry2

        lax.fori_loop(0, IB // NBUF - 1, group, 0)

        ce = IB - NBUF
        for b in range(NBUF):
            pltpu.make_async_copy(
                x_hbm.at[sidx.at[ce + b]], rows[b], gsem[b]).wait()
            pltpu.async_copy(
                rows[b], acc.at[didx.at[ce + b]], ssem[b], add=True)
        for b in range(NBUF):
            pltpu.make_async_copy(
                rows[b], acc.at[didx.at[ce + b]], ssem[b]).wait()
        return carry

    lax.fori_loop(0, NB, batch, 0)
    plsc.subcore_barrier()
    pltpu.sync_copy(acc.at[pl.ds(base, RPT)], out_hbm.at[c, pl.ds(base, RPT)])


_sc_agg = pl.kernel(
    _sc_agg_body,
    out_type=jax.ShapeDtypeStruct((NC, N_PAD, B), jnp.float32),
    mesh=_MESH,
    compiler_params=_SC_PARAMS,
    scratch_types=[
        pltpu.VMEM((IB, CH), jnp.int32),
        pltpu.VMEM((IB, CH), jnp.int32),
        pltpu.VMEM((CH, B), jnp.float32),
        pltpu.VMEM((CH, B), jnp.float32),
        pltpu.VMEM((CH, B), jnp.float32),
        pltpu.VMEM((CH, B), jnp.float32),
        pltpu.VMEM((ZRS, B), jnp.float32),
        pltpu.VMEM_SHARED((N_PAD, B), jnp.float32),
        pltpu.SemaphoreType.DMA,
        pltpu.SemaphoreType.DMA,
        pltpu.SemaphoreType.DMA,
        pltpu.SemaphoreType.DMA,
        pltpu.SemaphoreType.DMA,
        pltpu.SemaphoreType.DMA,
        pltpu.SemaphoreType.DMA,
        pltpu.SemaphoreType.DMA,
    ],
)


def _sc_deg_body(dst_hbm, out_hbm, didx, ones_r, zrow, acc):
    c = lax.axis_index("c")
    s = lax.axis_index("s")
    w = c * NS + s

    z16 = jnp.zeros((16,), jnp.float32)
    o16 = jnp.ones((16,), jnp.float32)

    def ofill(i, carry):
        ones_r[i, 0:16] = o16
        return carry

    lax.fori_loop(0, CH, ofill, 0)

    def zfill(i, carry):
        zrow[i, 0:16] = z16
        return carry

    lax.fori_loop(0, ZR, zfill, 0)

    base = s * RPT
    for j in range(RPT // ZR):
        pltpu.sync_copy(zrow, acc.at[pl.ds(base + j * ZR, ZR)])

    pltpu.sync_copy(dst_hbm.at[pl.ds(w * NCH, NCH)], didx)
    plsc.subcore_barrier()

    def chunk(ci, carry):
        pltpu.sync_copy(ones_r, acc.at[didx.at[ci]], add=True)
        return carry

    lax.fori_loop(0, NCH, chunk, 0)
    plsc.subcore_barrier()
    pltpu.sync_copy(acc.at[pl.ds(base, RPT)], out_hbm.at[c, pl.ds(base, RPT)])


_sc_deg = pl.kernel(
    _sc_deg_body,
    out_type=jax.ShapeDtypeStruct((NC, N_PAD, DW), jnp.float32),
    mesh=_MESH,
    compiler_params=_SC_PARAMS,
    scratch_types=[
        pltpu.VMEM((NCH, CH), jnp.int32),
        pltpu.VMEM((CH, DW), jnp.float32),
        pltpu.VMEM((ZR, DW), jnp.float32),
        pltpu.VMEM_SHARED((N_PAD, DW), jnp.float32),
    ],
)


def _bs2():
    return pl.BlockSpec((RB, B), lambda i: (i, 0))


def _bs1():
    return pl.BlockSpec((RB, 1), lambda i: (i, 0))


def _bsa():
    return pl.BlockSpec((NC, RB, B), lambda i: (0, i, 0))


def _f2(shape=(N_PAD, B)):
    return jax.ShapeDtypeStruct(shape, jnp.float32)


TB = 2176


def _tx_body(xb_ref, xo_ref):
    xo_ref[...] = jnp.transpose(xb_ref[...])


_tx = pl.pallas_call(
    _tx_body,
    grid=(N_PAD // TB,),
    in_specs=[pl.BlockSpec((B, TB), lambda i: (0, i))],
    out_specs=pl.BlockSpec((TB, B), lambda i: (i, 0)),
    out_shape=jax.ShapeDtypeStruct((N_PAD, B), jnp.float32),
)


def _prep_body(ab_ref, deg_ref, gate_ref, a_ref, ga_ref, gb_ref):
    d = jnp.maximum(deg_ref[0, :, 0:1] + deg_ref[1, :, 0:1], 1.0)
    g = gate_ref[...]
    m = jnp.max(g, axis=1, keepdims=True)
    e = jnp.exp(g - m)
    tot = e[:, 0:1] + e[:, 1:2]
    g0 = e[:, 0:1] / tot
    g1 = e[:, 1:2] / tot
    ga = g0 * ab_ref[0, 0] + g1 * ab_ref[0, 1]
    gb = g0 * ab_ref[1, 0] + g1 * ab_ref[1, 1]
    a_ref[...] = ga / d
    ga_ref[...] = ga
    gb_ref[...] = gb


_prep = pl.pallas_call(
    _prep_body,
    grid=(N_PAD // RB,),
    in_specs=[
        pl.BlockSpec(memory_space=pltpu.SMEM),
        pl.BlockSpec((NC, RB, DW), lambda i: (0, i, 0)),
        pl.BlockSpec((RB, K), lambda i: (i, 0)),
    ],
    out_specs=[_bs1(), _bs1(), _bs1()],
    out_shape=[_f2((N_PAD, 1))] * 3,
)


def _k_of(agg_ref, y, a_ref, ga_ref, gb_ref):
    agg = agg_ref[0] + agg_ref[1]
    return a_ref[...] * agg - ga_ref[...] * y + gb_ref[...] * (y - y * y)


def _stage1_body(agg_ref, y_ref, a_ref, ga_ref, gb_ref, yo_ref, ko_ref):
    y = y_ref[...]
    k = _k_of(agg_ref, y, a_ref, ga_ref, gb_ref)
    yo_ref[...] = y + 0.5 * k
    ko_ref[...] = k


_stage1 = pl.pallas_call(
    _stage1_body,
    grid=(N_PAD // RB,),
    in_specs=[_bsa(), _bs2(), _bs1(), _bs1(), _bs1()],
    out_specs=[_bs2(), _bs2()],
    out_shape=[_f2(), _f2()],
)


def _mid_body(wk, cy, agg_ref, y_ref, x0_ref, acc_ref, a_ref, ga_ref, gb_ref,
              yo_ref, ao_ref):
    y = y_ref[...]
    k = _k_of(agg_ref, y, a_ref, ga_ref, gb_ref)
    yo_ref[...] = x0_ref[...] + cy * k
    ao_ref[...] = acc_ref[...] + wk * k


def _mk_mid(wk, cy):
    return pl.pallas_call(
        functools.partial(_mid_body, wk, cy),
        grid=(N_PAD // RB,),
        in_specs=[_bsa(), _bs2(), _bs2(), _bs2(), _bs1(), _bs1(), _bs1()],
        out_specs=[_bs2(), _bs2()],
        out_shape=[_f2(), _f2()],
    )


_mid2 = _mk_mid(2.0, 0.5)
_mid3 = _mk_mid(2.0, 1.0)


def _final_body(agg_ref, y_ref, x0_ref, acc_ref, a_ref, ga_ref, gb_ref,
                xo_ref):
    y = y_ref[...]
    k = _k_of(agg_ref, y, a_ref, ga_ref, gb_ref)
    xo_ref[...] = x0_ref[...] + (acc_ref[...] + k) * (1.0 / 6.0)


_final = pl.pallas_call(
    _final_body,
    grid=(N_PAD // RB,),
    in_specs=[_bsa(), _bs2(), _bs2(), _bs2(), _bs1(), _bs1(), _bs1()],
    out_specs=_bs2(),
    out_shape=_f2(),
)


def kernel(inputs, gate, edge_index, alpha, beta):
    x_bn = jnp.pad(inputs[:, 0, :, -1], ((0, 0), (0, N_PAD - N)))
    x0 = _tx(x_bn)
    gate_p = jnp.pad(gate, ((0, N_PAD - N), (0, 0)))
    src = edge_index[0]
    dst = edge_index[1]
    pad = E_PAD - E
    src_r = jnp.concatenate(
        [src, jnp.zeros((pad,), jnp.int32)]).reshape(E_PAD // CH, CH)
    dst_r = jnp.concatenate(
        [dst, jnp.full((pad,), DUMP, jnp.int32)]).reshape(E_PAD // CH, CH)

    deg_p = _sc_deg(dst_r)
    ab = jnp.stack([alpha, beta])
    a_s, ga_s, gb_s = _prep(ab, deg_p, gate_p)

    agg = _sc_agg(x0, src_r, dst_r)
    y2, acc = _stage1(agg, x0, a_s, ga_s, gb_s)
    agg = _sc_agg(y2, src_r, dst_r)
    y3, acc = _mid2(agg, y2, x0, acc, a_s, ga_s, gb_s)
    agg = _sc_agg(y3, src_r, dst_r)
    y4, acc = _mid3(agg, y3, x0, acc, a_s, ga_s, gb_s)
    agg = _sc_agg(y4, src_r, dst_r)
    xf = _final(agg, y4, x0, acc, a_s, ga_s, gb_s)
    return jnp.transpose(xf[:N])[None]


# speedup vs baseline: 1.0671x; 1.0671x over previous
import functools

import jax
import jax.numpy as jnp
from jax import lax
from jax.experimental import pallas as pl
from jax.experimental.pallas import tpu as pltpu
from jax.experimental.pallas import tpu_sc as plsc

N = 50000
N_PAD = 50048
B = 32
E = 800000
K = 2
NC = 2
NS = 16
NW = NC * NS
CH = 128
E_PAD = 819200
EW = E_PAD // NW
NCH = EW // CH
RPT = N_PAD // NS
DUMP = N
DW = 16
ZR = 782
IB = 40
NB = NCH // IB
ZRS = 92
NBUF = 4
RB = 3128

_MESH = plsc.VectorSubcoreMesh(core_axis_name="c", subcore_axis_name="s")
_SC_PARAMS = pltpu.CompilerParams(use_tc_tiling_on_sc=False)


def _sc_agg_body(x_hbm, src_hbm, dst_hbm, out_hbm, sidx, didx, rows0, rows1,
                 rows2, rows3, zrow, acc, gsem0, gsem1, gsem2, gsem3):
    rows = (rows0, rows1, rows2, rows3)
    gsem = (gsem0, gsem1, gsem2, gsem3)
    c = lax.axis_index("c")
    s = lax.axis_index("s")
    w = c * NS + s

    z16 = jnp.zeros((16,), jnp.float32)

    def zfill(i, carry):
        zrow[i, 0:16] = z16
        zrow[i, 16:32] = z16
        return carry

    lax.fori_loop(0, ZRS, zfill, 0)

    base = s * RPT
    for j in range(RPT // ZRS):
        pltpu.sync_copy(zrow, acc.at[pl.ds(base + j * ZRS, ZRS)])

    plsc.subcore_barrier()

    def batch(bi, carry):
        pltpu.sync_copy(src_hbm.at[pl.ds(w * NCH + bi * IB, IB)], sidx)
        pltpu.sync_copy(dst_hbm.at[pl.ds(w * NCH + bi * IB, IB)], didx)

        for b in range(NBUF):
            pltpu.async_copy(x_hbm.at[sidx.at[b]], rows[b], gsem[b])

        def group(g, carry2):
            c = NBUF * g
            for b in range(NBUF):
                pltpu.make_async_copy(
                    x_hbm.at[sidx.at[c + b]], rows[b], gsem[b]).wait()
                pltpu.sync_copy(rows[b], acc.at[didx.at[c + b]], add=True)
                pltpu.async_copy(
                    x_hbm.at[sidx.at[c + NBUF + b]], rows[b], gsem[b])
            return carry2

        lax.fori_loop(0, IB // NBUF - 1, group, 0)

        ce = IB - NBUF
        for b in range(NBUF):
            pltpu.make_async_copy(
                x_hbm.at[sidx.at[ce + b]], rows[b], gsem[b]).wait()
            pltpu.sync_copy(rows[b], acc.at[didx.at[ce + b]], add=True)
        return carry

    lax.fori_loop(0, NB, batch, 0)
    plsc.subcore_barrier()
    pltpu.sync_copy(acc.at[pl.ds(base, RPT)], out_hbm.at[c, pl.ds(base, RPT)])


_sc_agg = pl.kernel(
    _sc_agg_body,
    out_type=jax.ShapeDtypeStruct((NC, N_PAD, B), jnp.float32),
    mesh=_MESH,
    compiler_params=_SC_PARAMS,
    scratch_types=[
        pltpu.VMEM((IB, CH), jnp.int32),
        pltpu.VMEM((IB, CH), jnp.int32),
        pltpu.VMEM((CH, B), jnp.float32),
        pltpu.VMEM((CH, B), jnp.float32),
        pltpu.VMEM((CH, B), jnp.float32),
        pltpu.VMEM((CH, B), jnp.float32),
        pltpu.VMEM((ZRS, B), jnp.float32),
        pltpu.VMEM_SHARED((N_PAD, B), jnp.float32),
        pltpu.SemaphoreType.DMA,
        pltpu.SemaphoreType.DMA,
        pltpu.SemaphoreType.DMA,
        pltpu.SemaphoreType.DMA,
    ],
)


def _sc_deg_body(dst_hbm, out_hbm, didx, ones_r, zrow, acc):
    c = lax.axis_index("c")
    s = lax.axis_index("s")
    w = c * NS + s

    z16 = jnp.zeros((16,), jnp.float32)
    o16 = jnp.ones((16,), jnp.float32)

    def ofill(i, carry):
        ones_r[i, 0:16] = o16
        return carry

    lax.fori_loop(0, CH, ofill, 0)

    def zfill(i, carry):
        zrow[i, 0:16] = z16
        return carry

    lax.fori_loop(0, ZR, zfill, 0)

    base = s * RPT
    for j in range(RPT // ZR):
        pltpu.sync_copy(zrow, acc.at[pl.ds(base + j * ZR, ZR)])

    pltpu.sync_copy(dst_hbm.at[pl.ds(w * NCH, NCH)], didx)
    plsc.subcore_barrier()

    def chunk(ci, carry):
        pltpu.sync_copy(ones_r, acc.at[didx.at[ci]], add=True)
        return carry

    lax.fori_loop(0, NCH, chunk, 0)
    plsc.subcore_barrier()
    pltpu.sync_copy(acc.at[pl.ds(base, RPT)], out_hbm.at[c, pl.ds(base, RPT)])


_sc_deg = pl.kernel(
    _sc_deg_body,
    out_type=jax.ShapeDtypeStruct((NC, N_PAD, DW), jnp.float32),
    mesh=_MESH,
    compiler_params=_SC_PARAMS,
    scratch_types=[
        pltpu.VMEM((NCH, CH), jnp.int32),
        pltpu.VMEM((CH, DW), jnp.float32),
        pltpu.VMEM((ZR, DW), jnp.float32),
        pltpu.VMEM_SHARED((N_PAD, DW), jnp.float32),
    ],
)


def _bs2():
    return pl.BlockSpec((RB, B), lambda i: (i, 0))


def _bs1():
    return pl.BlockSpec((RB, 1), lambda i: (i, 0))


def _bsa():
    return pl.BlockSpec((NC, RB, B), lambda i: (0, i, 0))


def _f2(shape=(N_PAD, B)):
    return jax.ShapeDtypeStruct(shape, jnp.float32)


TB = 2176


def _tx_body(xb_ref, xo_ref):
    xo_ref[...] = jnp.transpose(xb_ref[...])


_tx = pl.pallas_call(
    _tx_body,
    grid=(N_PAD // TB,),
    in_specs=[pl.BlockSpec((B, TB), lambda i: (0, i))],
    out_specs=pl.BlockSpec((TB, B), lambda i: (i, 0)),
    out_shape=jax.ShapeDtypeStruct((N_PAD, B), jnp.float32),
)


def _prep_body(ab_ref, deg_ref, gate_ref, a_ref, ga_ref, gb_ref):
    d = jnp.maximum(deg_ref[0, :, 0:1] + deg_ref[1, :, 0:1], 1.0)
    g = gate_ref[...]
    m = jnp.max(g, axis=1, keepdims=True)
    e = jnp.exp(g - m)
    tot = e[:, 0:1] + e[:, 1:2]
    g0 = e[:, 0:1] / tot
    g1 = e[:, 1:2] / tot
    ga = g0 * ab_ref[0, 0] + g1 * ab_ref[0, 1]
    gb = g0 * ab_ref[1, 0] + g1 * ab_ref[1, 1]
    a_ref[...] = ga / d
    ga_ref[...] = ga
    gb_ref[...] = gb


_prep = pl.pallas_call(
    _prep_body,
    grid=(N_PAD // RB,),
    in_specs=[
        pl.BlockSpec(memory_space=pltpu.SMEM),
        pl.BlockSpec((NC, RB, DW), lambda i: (0, i, 0)),
        pl.BlockSpec((RB, K), lambda i: (i, 0)),
    ],
    out_specs=[_bs1(), _bs1(), _bs1()],
    out_shape=[_f2((N_PAD, 1))] * 3,
)


def _k_of(agg_ref, y, a_ref, ga_ref, gb_ref):
    agg = agg_ref[0] + agg_ref[1]
    return a_ref[...] * agg - ga_ref[...] * y + gb_ref[...] * (y - y * y)


def _stage1_body(agg_ref, y_ref, a_ref, ga_ref, gb_ref, yo_ref, ko_ref):
    y = y_ref[...]
    k = _k_of(agg_ref, y, a_ref, ga_ref, gb_ref)
    yo_ref[...] = y + 0.5 * k
    ko_ref[...] = k


_stage1 = pl.pallas_call(
    _stage1_body,
    grid=(N_PAD // RB,),
    in_specs=[_bsa(), _bs2(), _bs1(), _bs1(), _bs1()],
    out_specs=[_bs2(), _bs2()],
    out_shape=[_f2(), _f2()],
)


def _mid_body(wk, cy, agg_ref, y_ref, x0_ref, acc_ref, a_ref, ga_ref, gb_ref,
              yo_ref, ao_ref):
    y = y_ref[...]
    k = _k_of(agg_ref, y, a_ref, ga_ref, gb_ref)
    yo_ref[...] = x0_ref[...] + cy * k
    ao_ref[...] = acc_ref[...] + wk * k


def _mk_mid(wk, cy):
    return pl.pallas_call(
        functools.partial(_mid_body, wk, cy),
        grid=(N_PAD // RB,),
        in_specs=[_bsa(), _bs2(), _bs2(), _bs2(), _bs1(), _bs1(), _bs1()],
        out_specs=[_bs2(), _bs2()],
        out_shape=[_f2(), _f2()],
    )


_mid2 = _mk_mid(2.0, 0.5)
_mid3 = _mk_mid(2.0, 1.0)


def _final_body(agg_ref, y_ref, x0_ref, acc_ref, a_ref, ga_ref, gb_ref,
                xo_ref):
    y = y_ref[...]
    k = _k_of(agg_ref, y, a_ref, ga_ref, gb_ref)
    xo_ref[...] = x0_ref[...] + (acc_ref[...] + k) * (1.0 / 6.0)


_final = pl.pallas_call(
    _final_body,
    grid=(N_PAD // RB,),
    in_specs=[_bsa(), _bs2(), _bs2(), _bs2(), _bs1(), _bs1(), _bs1()],
    out_specs=_bs2(),
    out_shape=_f2(),
)


def kernel(inputs, gate, edge_index, alpha, beta):
    x_bn = jnp.pad(inputs[:, 0, :, -1], ((0, 0), (0, N_PAD - N)))
    x0 = _tx(x_bn)
    gate_p = jnp.pad(gate, ((0, N_PAD - N), (0, 0)))
    src = edge_index[0]
    dst = edge_index[1]
    pad = E_PAD - E
    src_r = jnp.concatenate(
        [src, jnp.zeros((pad,), jnp.int32)]).reshape(E_PAD // CH, CH)
    dst_r = jnp.concatenate(
        [dst, jnp.full((pad,), DUMP, jnp.int32)]).reshape(E_PAD // CH, CH)

    deg_p = _sc_deg(dst_r)
    ab = jnp.stack([alpha, beta])
    a_s, ga_s, gb_s = _prep(ab, deg_p, gate_p)

    agg = _sc_agg(x0, src_r, dst_r)
    y2, acc = _stage1(agg, x0, a_s, ga_s, gb_s)
    agg = _sc_agg(y2, src_r, dst_r)
    y3, acc = _mid2(agg, y2, x0, acc, a_s, ga_s, gb_s)
    agg = _sc_agg(y3, src_r, dst_r)
    y4, acc = _mid3(agg, y3, x0, acc, a_s, ga_s, gb_s)
    agg = _sc_agg(y4, src_r, dst_r)
    xf = _final(agg, y4, x0, acc, a_s, ga_s, gb_s)
    return jnp.transpose(xf[:N])[None]


# speedup vs baseline: 1.4370x; 1.3467x over previous
import functools

import jax
import jax.numpy as jnp
from jax import lax
from jax.experimental import pallas as pl
from jax.experimental.pallas import tpu as pltpu
from jax.experimental.pallas import tpu_sc as plsc

N = 50000
N_PAD = 50048
B = 32
E = 800000
K = 2
NC = 2
NS = 16
NW = NC * NS
CH = 128
E_PAD = 819200
EW = E_PAD // NW
NCH = EW // CH
RPT = N_PAD // NS
DUMP = N
DW = 16
ZR = 782
NBUF = 4
RB = 3128

_MESH = plsc.VectorSubcoreMesh(core_axis_name="c", subcore_axis_name="s")
_SC_PARAMS = pltpu.CompilerParams(use_tc_tiling_on_sc=False)


def _sc_agg_body(x_hbm, src_hbm, dst_hbm, z_hbm, out_hbm, sidx, didx, rows0,
                 rows1, rows2, rows3, zrow, acc, gsem0, gsem1, gsem2, gsem3):
    rows = (rows0, rows1, rows2, rows3)
    gsem = (gsem0, gsem1, gsem2, gsem3)
    c = lax.axis_index("c")
    s = lax.axis_index("s")
    w = c * NS + s

    pltpu.sync_copy(z_hbm, zrow)
    base = s * RPT
    for j in range(RPT // ZR):
        pltpu.sync_copy(zrow, acc.at[pl.ds(base + j * ZR, ZR)])

    pltpu.sync_copy(src_hbm.at[pl.ds(w * NCH, NCH)], sidx)
    pltpu.sync_copy(dst_hbm.at[pl.ds(w * NCH, NCH)], didx)
    plsc.subcore_barrier()

    for b in range(NBUF):
        pltpu.async_copy(x_hbm.at[sidx.at[b]], rows[b], gsem[b])

    def group(g, carry2):
        ci = NBUF * g
        for b in range(NBUF):
            pltpu.make_async_copy(
                x_hbm.at[sidx.at[ci + b]], rows[b], gsem[b]).wait()
            pltpu.sync_copy(rows[b], acc.at[didx.at[ci + b]], add=True)
            pltpu.async_copy(
                x_hbm.at[sidx.at[ci + NBUF + b]], rows[b], gsem[b])
        return carry2

    lax.fori_loop(0, NCH // NBUF - 1, group, 0)

    ce = NCH - NBUF
    for b in range(NBUF):
        pltpu.make_async_copy(
            x_hbm.at[sidx.at[ce + b]], rows[b], gsem[b]).wait()
        pltpu.sync_copy(rows[b], acc.at[didx.at[ce + b]], add=True)

    plsc.subcore_barrier()
    pltpu.sync_copy(acc.at[pl.ds(base, RPT)], out_hbm.at[c, pl.ds(base, RPT)])


_sc_agg = pl.kernel(
    _sc_agg_body,
    out_type=jax.ShapeDtypeStruct((NC, N_PAD, B), jnp.bfloat16),
    mesh=_MESH,
    compiler_params=_SC_PARAMS,
    scratch_types=[
        pltpu.VMEM((NCH, CH), jnp.int32),
        pltpu.VMEM((NCH, CH), jnp.int32),
        pltpu.VMEM((CH, B), jnp.bfloat16),
        pltpu.VMEM((CH, B), jnp.bfloat16),
        pltpu.VMEM((CH, B), jnp.bfloat16),
        pltpu.VMEM((CH, B), jnp.bfloat16),
        pltpu.VMEM((ZR, B), jnp.bfloat16),
        pltpu.VMEM_SHARED((N_PAD, B), jnp.bfloat16),
        pltpu.SemaphoreType.DMA,
        pltpu.SemaphoreType.DMA,
        pltpu.SemaphoreType.DMA,
        pltpu.SemaphoreType.DMA,
    ],
)


def _sc_deg_body(dst_hbm, out_hbm, didx, ones_r, zrow, acc):
    c = lax.axis_index("c")
    s = lax.axis_index("s")
    w = c * NS + s

    z16 = jnp.zeros((16,), jnp.float32)
    o16 = jnp.ones((16,), jnp.float32)

    def ofill(i, carry):
        ones_r[i, 0:16] = o16
        return carry

    lax.fori_loop(0, CH, ofill, 0)

    def zfill(i, carry):
        zrow[i, 0:16] = z16
        return carry

    lax.fori_loop(0, ZR, zfill, 0)

    base = s * RPT
    for j in range(RPT // ZR):
        pltpu.sync_copy(zrow, acc.at[pl.ds(base + j * ZR, ZR)])

    pltpu.sync_copy(dst_hbm.at[pl.ds(w * NCH, NCH)], didx)
    plsc.subcore_barrier()

    def chunk(ci, carry):
        pltpu.sync_copy(ones_r, acc.at[didx.at[ci]], add=True)
        return carry

    lax.fori_loop(0, NCH, chunk, 0)
    plsc.subcore_barrier()
    pltpu.sync_copy(acc.at[pl.ds(base, RPT)], out_hbm.at[c, pl.ds(base, RPT)])


_sc_deg = pl.kernel(
    _sc_deg_body,
    out_type=jax.ShapeDtypeStruct((NC, N_PAD, DW), jnp.float32),
    mesh=_MESH,
    compiler_params=_SC_PARAMS,
    scratch_types=[
        pltpu.VMEM((NCH, CH), jnp.int32),
        pltpu.VMEM((CH, DW), jnp.float32),
        pltpu.VMEM((ZR, DW), jnp.float32),
        pltpu.VMEM_SHARED((N_PAD, DW), jnp.float32),
    ],
)


def _bs2():
    return pl.BlockSpec((RB, B), lambda i: (i, 0))


def _bs1():
    return pl.BlockSpec((RB, 1), lambda i: (i, 0))


def _bsa():
    return pl.BlockSpec((NC, RB, B), lambda i: (0, i, 0))


def _f2(shape=(N_PAD, B)):
    return jax.ShapeDtypeStruct(shape, jnp.float32)


TB = 2176


def _tx_body(xb_ref, xo_ref, xb16_ref):
    xt = jnp.transpose(xb_ref[...])
    xo_ref[...] = xt
    xb16_ref[...] = xt.astype(jnp.bfloat16)


_tx = pl.pallas_call(
    _tx_body,
    grid=(N_PAD // TB,),
    in_specs=[pl.BlockSpec((B, TB), lambda i: (0, i))],
    out_specs=[pl.BlockSpec((TB, B), lambda i: (i, 0))] * 2,
    out_shape=[jax.ShapeDtypeStruct((N_PAD, B), jnp.float32),
               jax.ShapeDtypeStruct((N_PAD, B), jnp.bfloat16)],
)


def _prep_body(ab_ref, deg_ref, gate_ref, a_ref, ga_ref, gb_ref):
    d = jnp.maximum(deg_ref[0, :, 0:1] + deg_ref[1, :, 0:1], 1.0)
    g = gate_ref[...]
    m = jnp.max(g, axis=1, keepdims=True)
    e = jnp.exp(g - m)
    tot = e[:, 0:1] + e[:, 1:2]
    g0 = e[:, 0:1] / tot
    g1 = e[:, 1:2] / tot
    ga = g0 * ab_ref[0, 0] + g1 * ab_ref[0, 1]
    gb = g0 * ab_ref[1, 0] + g1 * ab_ref[1, 1]
    a_ref[...] = ga / d
    ga_ref[...] = ga
    gb_ref[...] = gb


_prep = pl.pallas_call(
    _prep_body,
    grid=(N_PAD // RB,),
    in_specs=[
        pl.BlockSpec(memory_space=pltpu.SMEM),
        pl.BlockSpec((NC, RB, DW), lambda i: (0, i, 0)),
        pl.BlockSpec((RB, K), lambda i: (i, 0)),
    ],
    out_specs=[_bs1(), _bs1(), _bs1()],
    out_shape=[_f2((N_PAD, 1))] * 3,
)


def _fb(shape=(N_PAD, B)):
    return jax.ShapeDtypeStruct(shape, jnp.bfloat16)


def _k_of(agg_ref, y, a_ref, ga_ref, gb_ref):
    agg = (agg_ref[0].astype(jnp.float32)
           + agg_ref[1].astype(jnp.float32))
    return a_ref[...] * agg - ga_ref[...] * y + gb_ref[...] * (y - y * y)


def _stage1_body(agg_ref, y_ref, a_ref, ga_ref, gb_ref, yo_ref, yb_ref,
                 ko_ref):
    y = y_ref[...]
    k = _k_of(agg_ref, y, a_ref, ga_ref, gb_ref)
    yn = y + 0.5 * k
    yo_ref[...] = yn
    yb_ref[...] = yn.astype(jnp.bfloat16)
    ko_ref[...] = k


_stage1 = pl.pallas_call(
    _stage1_body,
    grid=(N_PAD // RB,),
    in_specs=[_bsa(), _bs2(), _bs1(), _bs1(), _bs1()],
    out_specs=[_bs2(), _bs2(), _bs2()],
    out_shape=[_f2(), _fb(), _f2()],
)


def _mid_body(wk, cy, agg_ref, y_ref, x0_ref, acc_ref, a_ref, ga_ref, gb_ref,
              yo_ref, yb_ref, ao_ref):
    y = y_ref[...]
    k = _k_of(agg_ref, y, a_ref, ga_ref, gb_ref)
    yn = x0_ref[...] + cy * k
    yo_ref[...] = yn
    yb_ref[...] = yn.astype(jnp.bfloat16)
    ao_ref[...] = acc_ref[...] + wk * k


def _mk_mid(wk, cy):
    return pl.pallas_call(
        functools.partial(_mid_body, wk, cy),
        grid=(N_PAD // RB,),
        in_specs=[_bsa(), _bs2(), _bs2(), _bs2(), _bs1(), _bs1(), _bs1()],
        out_specs=[_bs2(), _bs2(), _bs2()],
        out_shape=[_f2(), _fb(), _f2()],
    )


_mid2 = _mk_mid(2.0, 0.5)
_mid3 = _mk_mid(2.0, 1.0)


def _final_body(agg_ref, y_ref, x0_ref, acc_ref, a_ref, ga_ref, gb_ref,
                xo_ref):
    y = y_ref[...]
    k = _k_of(agg_ref, y, a_ref, ga_ref, gb_ref)
    xo_ref[...] = x0_ref[...] + (acc_ref[...] + k) * (1.0 / 6.0)


_final = pl.pallas_call(
    _final_body,
    grid=(N_PAD // RB,),
    in_specs=[_bsa(), _bs2(), _bs2(), _bs2(), _bs1(), _bs1(), _bs1()],
    out_specs=_bs2(),
    out_shape=_f2(),
)


def kernel(inputs, gate, edge_index, alpha, beta):
    x_bn = jnp.pad(inputs[:, 0, :, -1], ((0, 0), (0, N_PAD - N)))
    x0, x0b = _tx(x_bn)
    zrow = jnp.zeros((ZR, B), jnp.bfloat16)
    gate_p = jnp.pad(gate, ((0, N_PAD - N), (0, 0)))
    src = edge_index[0]
    dst = edge_index[1]
    pad = E_PAD - E
    src_r = jnp.concatenate(
        [src, jnp.zeros((pad,), jnp.int32)]).reshape(E_PAD // CH, CH)
    dst_r = jnp.concatenate(
        [dst, jnp.full((pad,), DUMP, jnp.int32)]).reshape(E_PAD // CH, CH)

    deg_p = _sc_deg(dst_r)
    ab = jnp.stack([alpha, beta])
    a_s, ga_s, gb_s = _prep(ab, deg_p, gate_p)

    agg = _sc_agg(x0b, src_r, dst_r, zrow)
    y2, y2b, acc = _stage1(agg, x0, a_s, ga_s, gb_s)
    agg = _sc_agg(y2b, src_r, dst_r, zrow)
    y3, y3b, acc = _mid2(agg, y2, x0, acc, a_s, ga_s, gb_s)
    agg = _sc_agg(y3b, src_r, dst_r, zrow)
    y4, y4b, acc = _mid3(agg, y3, x0, acc, a_s, ga_s, gb_s)
    agg = _sc_agg(y4b, src_r, dst_r, zrow)
    xf = _final(agg, y4, x0, acc, a_s, ga_s, gb_s)
    return jnp.transpose(xf[:N])[None]


# speedup vs baseline: 1.4564x; 1.0135x over previous
import functools

import jax
import jax.numpy as jnp
from jax import lax
from jax.experimental import pallas as pl
from jax.experimental.pallas import tpu as pltpu
from jax.experimental.pallas import tpu_sc as plsc

N = 50000
N_PAD = 50048
B = 32
E = 800000
K = 2
NC = 2
NS = 16
NW = NC * NS
CH = 128
E_PAD = 819200
EW = E_PAD // NW
NCH = EW // CH
RPT = N_PAD // NS
DUMP = N
DW = 16
ZR = 782
NBUF = 4
RB = 3128

_MESH = plsc.VectorSubcoreMesh(core_axis_name="c", subcore_axis_name="s")
_SC_PARAMS = pltpu.CompilerParams(use_tc_tiling_on_sc=False)


def _sc_agg_body(x_hbm, src_hbm, dst_hbm, z_hbm, out_hbm, sidx, didx, rows0,
                 rows1, rows2, rows3, zrow, acc, gsem0, gsem1, gsem2, gsem3):
    rows = (rows0, rows1, rows2, rows3)
    gsem = (gsem0, gsem1, gsem2, gsem3)
    c = lax.axis_index("c")
    s = lax.axis_index("s")
    w = c * NS + s

    pltpu.sync_copy(z_hbm, zrow)
    base = s * RPT
    for j in range(RPT // ZR):
        pltpu.sync_copy(zrow, acc.at[pl.ds(base + j * ZR, ZR)])

    pltpu.sync_copy(src_hbm.at[pl.ds(w * NCH, NCH)], sidx)
    pltpu.sync_copy(dst_hbm.at[pl.ds(w * NCH, NCH)], didx)
    plsc.subcore_barrier()

    for b in range(NBUF):
        pltpu.async_copy(x_hbm.at[sidx.at[b]], rows[b], gsem[b])

    def group(g, carry2):
        ci = NBUF * g
        for b in range(NBUF):
            pltpu.make_async_copy(
                x_hbm.at[sidx.at[ci + b]], rows[b], gsem[b]).wait()
            pltpu.sync_copy(rows[b], acc.at[didx.at[ci + b]], add=True)
            pltpu.async_copy(
                x_hbm.at[sidx.at[ci + NBUF + b]], rows[b], gsem[b])
        return carry2

    lax.fori_loop(0, NCH // NBUF - 1, group, 0)

    ce = NCH - NBUF
    for b in range(NBUF):
        pltpu.make_async_copy(
            x_hbm.at[sidx.at[ce + b]], rows[b], gsem[b]).wait()
        pltpu.sync_copy(rows[b], acc.at[didx.at[ce + b]], add=True)

    plsc.subcore_barrier()
    pltpu.sync_copy(acc.at[pl.ds(base, RPT)], out_hbm.at[c, pl.ds(base, RPT)])


_sc_agg = pl.kernel(
    _sc_agg_body,
    out_type=jax.ShapeDtypeStruct((NC, N_PAD, B), jnp.bfloat16),
    mesh=_MESH,
    compiler_params=_SC_PARAMS,
    scratch_types=[
        pltpu.VMEM((NCH, CH), jnp.int32),
        pltpu.VMEM((NCH, CH), jnp.int32),
        pltpu.VMEM((CH, B), jnp.bfloat16),
        pltpu.VMEM((CH, B), jnp.bfloat16),
        pltpu.VMEM((CH, B), jnp.bfloat16),
        pltpu.VMEM((CH, B), jnp.bfloat16),
        pltpu.VMEM((ZR, B), jnp.bfloat16),
        pltpu.VMEM_SHARED((N_PAD, B), jnp.bfloat16),
        pltpu.SemaphoreType.DMA,
        pltpu.SemaphoreType.DMA,
        pltpu.SemaphoreType.DMA,
        pltpu.SemaphoreType.DMA,
    ],
)


def _sc_deg_body(dst_hbm, out_hbm, didx, ones_r, zrow, acc):
    c = lax.axis_index("c")
    s = lax.axis_index("s")
    w = c * NS + s

    z16 = jnp.zeros((16,), jnp.float32)
    o16 = jnp.ones((16,), jnp.float32)

    def ofill(i, carry):
        ones_r[i, 0:16] = o16
        return carry

    lax.fori_loop(0, CH, ofill, 0)

    def zfill(i, carry):
        zrow[i, 0:16] = z16
        return carry

    lax.fori_loop(0, ZR, zfill, 0)

    base = s * RPT
    for j in range(RPT // ZR):
        pltpu.sync_copy(zrow, acc.at[pl.ds(base + j * ZR, ZR)])

    pltpu.sync_copy(dst_hbm.at[pl.ds(w * NCH, NCH)], didx)
    plsc.subcore_barrier()

    def chunk(ci, carry):
        pltpu.sync_copy(ones_r, acc.at[didx.at[ci]], add=True)
        return carry

    lax.fori_loop(0, NCH, chunk, 0)
    plsc.subcore_barrier()
    pltpu.sync_copy(acc.at[pl.ds(base, RPT)], out_hbm.at[c, pl.ds(base, RPT)])


_sc_deg = pl.kernel(
    _sc_deg_body,
    out_type=jax.ShapeDtypeStruct((NC, N_PAD, DW), jnp.float32),
    mesh=_MESH,
    compiler_params=_SC_PARAMS,
    scratch_types=[
        pltpu.VMEM((NCH, CH), jnp.int32),
        pltpu.VMEM((CH, DW), jnp.float32),
        pltpu.VMEM((ZR, DW), jnp.float32),
        pltpu.VMEM_SHARED((N_PAD, DW), jnp.float32),
    ],
)


def _bs2():
    return pl.BlockSpec((RB, B), lambda i: (i, 0))


def _bs1():
    return pl.BlockSpec((RB, 1), lambda i: (i, 0))


def _bsa():
    return pl.BlockSpec((NC, RB, B), lambda i: (0, i, 0))


def _f2(shape=(N_PAD, B)):
    return jax.ShapeDtypeStruct(shape, jnp.float32)


TB = 2176


def _tx_body(xb_ref, xo_ref, xb16_ref):
    xt = jnp.transpose(xb_ref[...])
    xo_ref[...] = xt
    xb16_ref[...] = xt.astype(jnp.bfloat16)


_tx = pl.pallas_call(
    _tx_body,
    grid=(N_PAD // TB,),
    in_specs=[pl.BlockSpec((B, TB), lambda i: (0, i))],
    out_specs=[pl.BlockSpec((TB, B), lambda i: (i, 0))] * 2,
    out_shape=[jax.ShapeDtypeStruct((N_PAD, B), jnp.float32),
               jax.ShapeDtypeStruct((N_PAD, B), jnp.bfloat16)],
)


def _prep_body(ab_ref, deg_ref, gate_ref, a_ref, ga_ref, gb_ref):
    d = jnp.maximum(deg_ref[0, :, 0:1] + deg_ref[1, :, 0:1], 1.0)
    g = gate_ref[...]
    m = jnp.max(g, axis=1, keepdims=True)
    e = jnp.exp(g - m)
    tot = e[:, 0:1] + e[:, 1:2]
    g0 = e[:, 0:1] / tot
    g1 = e[:, 1:2] / tot
    ga = g0 * ab_ref[0, 0] + g1 * ab_ref[0, 1]
    gb = g0 * ab_ref[1, 0] + g1 * ab_ref[1, 1]
    a_ref[...] = ga / d
    ga_ref[...] = ga
    gb_ref[...] = gb


_prep = pl.pallas_call(
    _prep_body,
    grid=(N_PAD // RB,),
    in_specs=[
        pl.BlockSpec(memory_space=pltpu.SMEM),
        pl.BlockSpec((NC, RB, DW), lambda i: (0, i, 0)),
        pl.BlockSpec((RB, K), lambda i: (i, 0)),
    ],
    out_specs=[_bs1(), _bs1(), _bs1()],
    out_shape=[_f2((N_PAD, 1))] * 3,
)


def _fb(shape=(N_PAD, B)):
    return jax.ShapeDtypeStruct(shape, jnp.bfloat16)


def _k_of(agg_ref, y, a_ref, ga_ref, gb_ref):
    agg = (agg_ref[0].astype(jnp.float32)
           + agg_ref[1].astype(jnp.float32))
    return a_ref[...] * agg - ga_ref[...] * y + gb_ref[...] * (y - y * y)


def _stage1_body(agg_ref, y_ref, a_ref, ga_ref, gb_ref, yo_ref, yb_ref,
                 ko_ref):
    y = y_ref[...]
    k = _k_of(agg_ref, y, a_ref, ga_ref, gb_ref)
    yn = y + 0.5 * k
    yo_ref[...] = yn
    yb_ref[...] = yn.astype(jnp.bfloat16)
    ko_ref[...] = k


_stage1 = pl.pallas_call(
    _stage1_body,
    grid=(N_PAD // RB,),
    in_specs=[_bsa(), _bs2(), _bs1(), _bs1(), _bs1()],
    out_specs=[_bs2(), _bs2(), _bs2()],
    out_shape=[_f2(), _fb(), _f2()],
)


def _mid_body(wk, cy, agg_ref, y_ref, x0_ref, acc_ref, a_ref, ga_ref, gb_ref,
              yo_ref, yb_ref, ao_ref):
    y = y_ref[...]
    k = _k_of(agg_ref, y, a_ref, ga_ref, gb_ref)
    yn = x0_ref[...] + cy * k
    yo_ref[...] = yn
    yb_ref[...] = yn.astype(jnp.bfloat16)
    ao_ref[...] = acc_ref[...] + wk * k


def _mk_mid(wk, cy):
    return pl.pallas_call(
        functools.partial(_mid_body, wk, cy),
        grid=(N_PAD // RB,),
        in_specs=[_bsa(), _bs2(), _bs2(), _bs2(), _bs1(), _bs1(), _bs1()],
        out_specs=[_bs2(), _bs2(), _bs2()],
        out_shape=[_f2(), _fb(), _f2()],
    )


_mid2 = _mk_mid(2.0, 0.5)
_mid3 = _mk_mid(2.0, 1.0)


def _final_body(agg_ref, y_ref, x0_ref, acc_ref, a_ref, ga_ref, gb_ref,
                xo_ref):
    y = y_ref[...]
    k = _k_of(agg_ref, y, a_ref, ga_ref, gb_ref)
    xo_ref[...] = x0_ref[...] + (acc_ref[...] + k) * (1.0 / 6.0)


_final = pl.pallas_call(
    _final_body,
    grid=(N_PAD // RB,),
    in_specs=[_bsa(), _bs2(), _bs2(), _bs2(), _bs1(), _bs1(), _bs1()],
    out_specs=_bs2(),
    out_shape=_f2(),
)


def kernel(inputs, gate, edge_index, alpha, beta):
    x_bn = jnp.pad(inputs[:, 0, :, -1], ((0, 0), (0, N_PAD - N)))
    x0, x0b = _tx(x_bn)
    zrow = jnp.zeros((ZR, B), jnp.bfloat16)
    gate_p = jnp.pad(gate, ((0, N_PAD - N), (0, 0)))
    src = edge_index[0]
    dst = edge_index[1]
    pad = E_PAD - E
    src_r = jnp.concatenate(
        [src, jnp.zeros((pad,), jnp.int32)]).reshape(E_PAD // CH, CH)
    pad_dst = DUMP + jnp.arange(pad, dtype=jnp.int32) % (N_PAD - N)
    dst_r = jnp.concatenate([dst, pad_dst]).reshape(E_PAD // CH, CH)

    deg_p = _sc_deg(dst_r)
    ab = jnp.stack([alpha, beta])
    a_s, ga_s, gb_s = _prep(ab, deg_p, gate_p)

    agg = _sc_agg(x0b, src_r, dst_r, zrow)
    y2, y2b, acc = _stage1(agg, x0, a_s, ga_s, gb_s)
    agg = _sc_agg(y2b, src_r, dst_r, zrow)
    y3, y3b, acc = _mid2(agg, y2, x0, acc, a_s, ga_s, gb_s)
    agg = _sc_agg(y3b, src_r, dst_r, zrow)
    y4, y4b, acc = _mid3(agg, y3, x0, acc, a_s, ga_s, gb_s)
    agg = _sc_agg(y4b, src_r, dst_r, zrow)
    xf = _final(agg, y4, x0, acc, a_s, ga_s, gb_s)
    return jnp.transpose(xf[:N])[None]
